# Initial kernel scaffold; baseline (speedup 1.0000x reference)
import jax
import jax.numpy as jnp
from jax.experimental import pallas as pl


def kernel(xyz, params):
    raise NotImplementedError("write your pallas kernel here")


# speedup vs baseline: 21.1798x; 21.1798x over previous
import functools

import numpy as np
import jax
import jax.numpy as jnp
from jax import lax
from jax.experimental import pallas as pl
from jax.experimental.pallas import tpu as pltpu
from jax.experimental.pallas import tpu_sc as plsc

_K = 24
_RT = 64
_TW = 128


def _knn_body(xp_ref, xt_ref, idx_ref):
    b = pl.program_id(0)
    x = xp_ref[0]
    xt = xt_ref[0]
    n = xt.shape[1]
    s = jnp.sum(x * x, axis=1, keepdims=True)
    st = jnp.sum(xt * xt, axis=0, keepdims=True)
    d = s + st - 2.0 * jnp.dot(x, xt, preferred_element_type=jnp.float32)
    lane = lax.broadcasted_iota(jnp.int32, d.shape, 1)
    cols = []
    for _ in range(_K):
        m = jnp.min(d, axis=1, keepdims=True)
        cand = jnp.where(d == m, lane, n)
        a = jnp.min(cand, axis=1, keepdims=True)
        cols.append(a)
        d = jnp.where(lane == a, jnp.float32(np.inf), d)
    idx_ref[0] = jnp.concatenate(cols, axis=1) + b * n


def _knn(xyz8, xyzT):
    B, N, _ = xyz8.shape
    grid = (B, N // _RT)
    return pl.pallas_call(
        _knn_body,
        grid=grid,
        in_specs=[
            pl.BlockSpec((1, _RT, 8), lambda b, r: (b, r, 0)),
            pl.BlockSpec((1, 8, N), lambda b, r: (b, 0, 0)),
        ],
        out_specs=pl.BlockSpec((1, _RT, _K), lambda b, r: (b, r, 0)),
        out_shape=jax.ShapeDtypeStruct((B, N, _K), jnp.int32),
    )(xyz8, xyzT)


def _gather_rows(table, idx):
    V, D = table.shape
    M = idx.shape[0]
    bw = M // 32
    ch = 768
    nsteps = bw // ch
    mesh = plsc.VectorSubcoreMesh(core_axis_name="c", subcore_axis_name="s")

    @functools.partial(
        pl.kernel, mesh=mesh,
        out_type=jax.ShapeDtypeStruct((M, D), jnp.float32),
        scratch_types=[
            pltpu.VMEM((ch,), jnp.int32),
            pltpu.VMEM((ch, D), jnp.float32),
            pltpu.SemaphoreType.DMA,
        ],
    )
    def gk(table_hbm, idx_hbm, out_hbm, idx_v, rows_v, sem):
        wid = lax.axis_index("s") * 2 + lax.axis_index("c")
        base = wid * bw

        def step(i, c):
            off = base + i * ch
            pltpu.sync_copy(idx_hbm.at[pl.ds(off, ch)], idx_v)
            pltpu.async_copy(table_hbm.at[idx_v], rows_v, sem).wait()
            pltpu.sync_copy(rows_v, out_hbm.at[pl.ds(off, ch)])
            return c

        lax.fori_loop(0, nsteps, step, 0)

    return gk(table, idx)


def _layer_body(tab_ref, g_ref, wq_ref, wk_ref, wv_ref, pw_ref,
                bq_ref, bk_ref, bv_ref, pb_ref,
                out_ref, sum_ref, ssq_ref, *, nb, d_out):
    f32 = jnp.float32
    feats = tab_ref[0]
    g = g_ref[0]
    q = jnp.dot(feats, wq_ref[...], preferred_element_type=f32) + bq_ref[...]
    ps = jnp.dot(feats, pw_ref[...], preferred_element_type=f32) + pb_ref[...]
    kk = jnp.dot(g, wk_ref[...], preferred_element_type=f32) + bk_ref[...]
    vv = jnp.dot(g, wv_ref[...], preferred_element_type=f32) + bv_ref[...]
    pn = jnp.dot(g, pw_ref[...], preferred_element_type=f32)
    k3 = kk.reshape(nb, _K, d_out)
    v3 = vv.reshape(nb, _K, d_out)
    p3 = ps[:, None, :] - pn.reshape(nb, _K, d_out)
    logits = (q[:, None, :] - k3 + p3) * f32(1.0 / np.sqrt(d_out))
    m = jnp.max(logits, axis=1, keepdims=True)
    e = jnp.exp(logits - m)
    den = jnp.sum(e, axis=1)
    o = jnp.sum(e * (v3 + p3), axis=1) / den
    out_ref[0] = o

    @pl.when((pl.program_id(0) == 0) & (pl.program_id(1) == 0))
    def _init():
        sum_ref[...] = jnp.zeros_like(sum_ref)
        ssq_ref[...] = jnp.zeros_like(ssq_ref)

    sum_ref[...] += jnp.sum(o, axis=0, keepdims=True)
    ssq_ref[...] += jnp.sum(o * o, axis=0, keepdims=True)


def _layer(tab, g, Wq, Wk, Wv, PW, bq, bk, bv, pb, d_out, nb):
    B, N, _ = tab.shape
    g3 = g.reshape(B, N * _K, _TW)
    grid = (B, N // nb)
    body = functools.partial(_layer_body, nb=nb, d_out=d_out)
    wspec = lambda shape: pl.BlockSpec(shape, lambda b, r: (0, 0))
    out, ssum, ssq = pl.pallas_call(
        body,
        grid=grid,
        in_specs=[
            pl.BlockSpec((1, nb, _TW), lambda b, r: (b, r, 0)),
            pl.BlockSpec((1, nb * _K, _TW), lambda b, r: (b, r, 0)),
            wspec((_TW, d_out)), wspec((_TW, d_out)), wspec((_TW, d_out)),
            wspec((_TW, d_out)),
            wspec((1, d_out)), wspec((1, d_out)), wspec((1, d_out)),
            wspec((1, d_out)),
        ],
        out_specs=[
            pl.BlockSpec((1, nb, d_out), lambda b, r: (b, r, 0)),
            pl.BlockSpec((1, d_out), lambda b, r: (0, 0)),
            pl.BlockSpec((1, d_out), lambda b, r: (0, 0)),
        ],
        out_shape=[
            jax.ShapeDtypeStruct((B, N, d_out), jnp.float32),
            jax.ShapeDtypeStruct((1, d_out), jnp.float32),
            jax.ShapeDtypeStruct((1, d_out), jnp.float32),
        ],
    )(tab, g3, Wq, Wk, Wv, PW, bq, bk, bv, pb)
    return out, ssum[0], ssq[0]


def _final_body(o4_ref, o2_ref, wr_ref, s4_ref, c_ref, out_ref):
    o4 = o4_ref[0]
    o2 = o2_ref[0]
    out_ref[0] = (o4 * s4_ref[...] +
                  jnp.dot(o2, wr_ref[...], preferred_element_type=jnp.float32)
                  + c_ref[...])


def _final(o4, o2, Wr, s4, c):
    B, N, F = o4.shape
    D2 = o2.shape[2]
    return pl.pallas_call(
        _final_body,
        grid=(B,),
        in_specs=[
            pl.BlockSpec((1, N, F), lambda b: (b, 0, 0)),
            pl.BlockSpec((1, N, D2), lambda b: (b, 0, 0)),
            pl.BlockSpec((D2, F), lambda b: (0, 0)),
            pl.BlockSpec((1, F), lambda b: (0, 0)),
            pl.BlockSpec((1, F), lambda b: (0, 0)),
        ],
        out_specs=pl.BlockSpec((1, N, F), lambda b: (b, 0, 0)),
        out_shape=jax.ShapeDtypeStruct((B, N, F), jnp.float32),
    )(o4, o2, Wr, s4, c)


def _fold_weights(p, sv, tv, xyz_off):
    f1w, f1b = p['fc1_w'], p['fc1_b']
    d_prev = f1w.shape[0]
    sw = sv[:, None] * f1w
    base_b = tv @ f1w + f1b

    def mk(wname, bname):
        W = sw @ p[wname]
        Wp = jnp.zeros((_TW, W.shape[1]), jnp.float32).at[:d_prev].set(W)
        bvec = base_b @ p[wname] + p[bname]
        return Wp, bvec[None, :]

    Wq, bq = mk('wq_w', 'wq_b')
    Wk, bk = mk('wk_w', 'wk_b')
    Wv, bv = mk('wv_w', 'wv_b')
    PW = jnp.zeros((_TW, p['pos_w'].shape[1]),
                   jnp.float32).at[xyz_off:xyz_off + 3].set(p['pos_w'])
    pb = p['pos_b'][None, :]
    return Wq, Wk, Wv, PW, bq, bk, bv, pb


def kernel(xyz, params):
    B, N, _ = xyz.shape
    f32 = jnp.float32
    xyz8 = jnp.pad(xyz, ((0, 0), (0, 0), (0, 5)))
    xyzT = jnp.swapaxes(xyz8, 1, 2)
    idx = _knn(xyz8, xyzT)
    flat_idx = idx.reshape(B * N * _K)

    douts = [8, 16, 32, 128]
    nbs = [128, 128, 128, 128]
    tab = jnp.pad(xyz, ((0, 0), (0, 0), (0, _TW - 3)))
    sv = jnp.ones((3,), f32)
    tv = jnp.zeros((3,), f32)
    xyz_off = 0
    o2_keep = s2 = t2 = None
    o4 = s4 = t4 = None
    for i in range(4):
        p = params['tl%d' % (i + 1)]
        Wq, Wk, Wv, PW, bq, bk, bv, pb = _fold_weights(p, sv, tv, xyz_off)
        g = _gather_rows(tab.reshape(B * N, _TW), flat_idx)
        o, ssum, ssq = _layer(tab, g, Wq, Wk, Wv, PW, bq, bk, bv, pb,
                              douts[i], nbs[i])
        mean = ssum / (B * N)
        var = ssq / (B * N) - mean * mean
        sv = params['bn%d_g' % (i + 1)] / jnp.sqrt(var + 1e-5)
        tv = params['bn%d_b' % (i + 1)] - mean * sv
        if i == 1:
            o2_keep, s2, t2 = o, sv, tv
        if i < 3:
            d = douts[i]
            tab = jnp.concatenate(
                [o, xyz, jnp.zeros((B, N, _TW - d - 3), f32)], axis=2)
            xyz_off = d
        else:
            o4, s4, t4 = o, sv, tv

    Wr = s2[:, None] * params['res_w']
    c = t4 + t2 @ params['res_w'] + params['res_b']
    f4 = _final(o4, o2_keep, Wr, s4[None, :], c[None, :])
    return (f4, f4, f4)


# speedup vs baseline: 26.3295x; 1.2431x over previous
import functools

import numpy as np
import jax
import jax.numpy as jnp
from jax import lax
from jax.experimental import pallas as pl
from jax.experimental.pallas import tpu as pltpu
from jax.experimental.pallas import tpu_sc as plsc

_K = 24
_RT = 128
_TW = 128


def _knn_body(xp_ref, xt_ref, idx_ref):
    b = pl.program_id(0)
    x = xp_ref[0]
    xt = xt_ref[0]
    n = xt.shape[1]
    s = jnp.sum(x * x, axis=1, keepdims=True)
    st = jnp.sum(xt * xt, axis=0, keepdims=True)
    d = s + st - 2.0 * jnp.dot(x, xt, preferred_element_type=jnp.float32)
    lane = lax.broadcasted_iota(jnp.int32, d.shape, 1)
    cols = []
    for _ in range(_K):
        m = jnp.min(d, axis=1, keepdims=True)
        cand = jnp.where(d == m, lane, n)
        a = jnp.min(cand, axis=1, keepdims=True)
        cols.append(a)
        d = jnp.where(lane == a, jnp.float32(np.inf), d)
    idx_ref[0] = jnp.concatenate(cols, axis=1) + b * n


def _knn(xyz8, xyzT):
    B, N, _ = xyz8.shape
    grid = (B, N // _RT)
    return pl.pallas_call(
        _knn_body,
        grid=grid,
        in_specs=[
            pl.BlockSpec((1, _RT, 8), lambda b, r: (b, r, 0)),
            pl.BlockSpec((1, 8, N), lambda b, r: (b, 0, 0)),
        ],
        out_specs=pl.BlockSpec((1, _RT, _K), lambda b, r: (b, r, 0)),
        out_shape=jax.ShapeDtypeStruct((B, N, _K), jnp.int32),
    )(xyz8, xyzT)


def _gather_rows(table, idx):
    V, D = table.shape
    M = idx.shape[0]
    bw = M // 32
    ch = 768
    nsteps = bw // ch
    mesh = plsc.VectorSubcoreMesh(core_axis_name="c", subcore_axis_name="s")

    @functools.partial(
        pl.kernel, mesh=mesh,
        out_type=jax.ShapeDtypeStruct((M, D), jnp.float32),
        scratch_types=[
            pltpu.VMEM((ch,), jnp.int32),
            pltpu.VMEM((ch, D), jnp.float32),
            pltpu.SemaphoreType.DMA,
        ],
    )
    def gk(table_hbm, idx_hbm, out_hbm, idx_v, rows_v, sem):
        wid = lax.axis_index("s") * 2 + lax.axis_index("c")
        base = wid * bw

        def step(i, c):
            off = base + i * ch
            pltpu.sync_copy(idx_hbm.at[pl.ds(off, ch)], idx_v)
            pltpu.async_copy(table_hbm.at[idx_v], rows_v, sem).wait()
            pltpu.sync_copy(rows_v, out_hbm.at[pl.ds(off, ch)])
            return c

        lax.fori_loop(0, nsteps, step, 0)

    return gk(table, idx)


def _layer_body(tab_ref, g_ref, wq_ref, wk_ref, wv_ref, pw_ref,
                bq_ref, bk_ref, bv_ref, pb_ref,
                out_ref, sum_ref, ssq_ref, *, nb, d_out):
    f32 = jnp.float32
    feats = tab_ref[0]
    g = g_ref[0]
    q = jnp.dot(feats, wq_ref[...], preferred_element_type=f32) + bq_ref[...]
    ps = jnp.dot(feats, pw_ref[...], preferred_element_type=f32) + pb_ref[...]
    kk = jnp.dot(g, wk_ref[...], preferred_element_type=f32) + bk_ref[...]
    vv = jnp.dot(g, wv_ref[...], preferred_element_type=f32) + bv_ref[...]
    pn = jnp.dot(g, pw_ref[...], preferred_element_type=f32)
    k3 = kk.reshape(nb, _K, d_out)
    v3 = vv.reshape(nb, _K, d_out)
    p3 = ps[:, None, :] - pn.reshape(nb, _K, d_out)
    logits = (q[:, None, :] - k3 + p3) * f32(1.0 / np.sqrt(d_out))
    m = jnp.max(logits, axis=1, keepdims=True)
    e = jnp.exp(logits - m)
    den = jnp.sum(e, axis=1)
    o = jnp.sum(e * (v3 + p3), axis=1) / den
    out_ref[0] = o

    @pl.when((pl.program_id(0) == 0) & (pl.program_id(1) == 0))
    def _init():
        sum_ref[...] = jnp.zeros_like(sum_ref)
        ssq_ref[...] = jnp.zeros_like(ssq_ref)

    sum_ref[...] += jnp.sum(o, axis=0, keepdims=True)
    ssq_ref[...] += jnp.sum(o * o, axis=0, keepdims=True)


def _layer(tab, g, Wq, Wk, Wv, PW, bq, bk, bv, pb, d_out, nb):
    B, N, _ = tab.shape
    g3 = g.reshape(B, N * _K, _TW)
    grid = (B, N // nb)
    body = functools.partial(_layer_body, nb=nb, d_out=d_out)
    wspec = lambda shape: pl.BlockSpec(shape, lambda b, r: (0, 0))
    out, ssum, ssq = pl.pallas_call(
        body,
        grid=grid,
        in_specs=[
            pl.BlockSpec((1, nb, _TW), lambda b, r: (b, r, 0)),
            pl.BlockSpec((1, nb * _K, _TW), lambda b, r: (b, r, 0)),
            wspec((_TW, d_out)), wspec((_TW, d_out)), wspec((_TW, d_out)),
            wspec((_TW, d_out)),
            wspec((1, d_out)), wspec((1, d_out)), wspec((1, d_out)),
            wspec((1, d_out)),
        ],
        out_specs=[
            pl.BlockSpec((1, nb, d_out), lambda b, r: (b, r, 0)),
            pl.BlockSpec((1, d_out), lambda b, r: (0, 0)),
            pl.BlockSpec((1, d_out), lambda b, r: (0, 0)),
        ],
        out_shape=[
            jax.ShapeDtypeStruct((B, N, d_out), jnp.float32),
            jax.ShapeDtypeStruct((1, d_out), jnp.float32),
            jax.ShapeDtypeStruct((1, d_out), jnp.float32),
        ],
    )(tab, g3, Wq, Wk, Wv, PW, bq, bk, bv, pb)
    return out, ssum[0], ssq[0]


def _final_body(o4_ref, o2_ref, wr_ref, s4_ref, c_ref, out_ref):
    o4 = o4_ref[0]
    o2 = o2_ref[0]
    out_ref[0] = (o4 * s4_ref[...] +
                  jnp.dot(o2, wr_ref[...], preferred_element_type=jnp.float32)
                  + c_ref[...])


def _final(o4, o2, Wr, s4, c):
    B, N, F = o4.shape
    D2 = o2.shape[2]
    return pl.pallas_call(
        _final_body,
        grid=(B,),
        in_specs=[
            pl.BlockSpec((1, N, F), lambda b: (b, 0, 0)),
            pl.BlockSpec((1, N, D2), lambda b: (b, 0, 0)),
            pl.BlockSpec((D2, F), lambda b: (0, 0)),
            pl.BlockSpec((1, F), lambda b: (0, 0)),
            pl.BlockSpec((1, F), lambda b: (0, 0)),
        ],
        out_specs=pl.BlockSpec((1, N, F), lambda b: (b, 0, 0)),
        out_shape=jax.ShapeDtypeStruct((B, N, F), jnp.float32),
    )(o4, o2, Wr, s4, c)


def _fold_weights(p, sv, tv, xyz_off):
    f1w, f1b = p['fc1_w'], p['fc1_b']
    d_prev = f1w.shape[0]
    sw = sv[:, None] * f1w
    base_b = tv @ f1w + f1b

    def mk(wname, bname):
        W = sw @ p[wname]
        Wp = jnp.zeros((_TW, W.shape[1]), jnp.float32).at[:d_prev].set(W)
        bvec = base_b @ p[wname] + p[bname]
        return Wp, bvec[None, :]

    Wq, bq = mk('wq_w', 'wq_b')
    Wk, bk = mk('wk_w', 'wk_b')
    Wv, bv = mk('wv_w', 'wv_b')
    PW = jnp.zeros((_TW, p['pos_w'].shape[1]),
                   jnp.float32).at[xyz_off:xyz_off + 3].set(p['pos_w'])
    pb = p['pos_b'][None, :]
    return Wq, Wk, Wv, PW, bq, bk, bv, pb


def kernel(xyz, params):
    B, N, _ = xyz.shape
    f32 = jnp.float32
    xyz8 = jnp.pad(xyz, ((0, 0), (0, 0), (0, 5)))
    xyzT = jnp.swapaxes(xyz8, 1, 2)
    idx = _knn(xyz8, xyzT)
    flat_idx = idx.reshape(B * N * _K)

    douts = [8, 16, 32, 128]
    nbs = [128, 128, 128, 128]
    tab = jnp.pad(xyz, ((0, 0), (0, 0), (0, _TW - 3)))
    sv = jnp.ones((3,), f32)
    tv = jnp.zeros((3,), f32)
    xyz_off = 0
    o2_keep = s2 = t2 = None
    o4 = s4 = t4 = None
    for i in range(4):
        p = params['tl%d' % (i + 1)]
        Wq, Wk, Wv, PW, bq, bk, bv, pb = _fold_weights(p, sv, tv, xyz_off)
        g = _gather_rows(tab.reshape(B * N, _TW), flat_idx)
        o, ssum, ssq = _layer(tab, g, Wq, Wk, Wv, PW, bq, bk, bv, pb,
                              douts[i], nbs[i])
        mean = ssum / (B * N)
        var = ssq / (B * N) - mean * mean
        sv = params['bn%d_g' % (i + 1)] / jnp.sqrt(var + 1e-5)
        tv = params['bn%d_b' % (i + 1)] - mean * sv
        if i == 1:
            o2_keep, s2, t2 = o, sv, tv
        if i < 3:
            d = douts[i]
            tab = jnp.concatenate(
                [o, xyz, jnp.zeros((B, N, _TW - d - 3), f32)], axis=2)
            xyz_off = d
        else:
            o4, s4, t4 = o, sv, tv

    Wr = s2[:, None] * params['res_w']
    c = t4 + t2 @ params['res_w'] + params['res_b']
    f4 = _final(o4, o2_keep, Wr, s4[None, :], c[None, :])
    return (f4, f4, f4)
